# Initial kernel scaffold; baseline (speedup 1.0000x reference)
import jax
import jax.numpy as jnp
from jax.experimental import pallas as pl


def kernel(hidden_states, gate_w, W_gate, W_up, W_down, sh_gate_w, sh_up_w, sh_down_w, shared_expert_gate_w):
    raise NotImplementedError("write your pallas kernel here")


# speedup vs baseline: 1.3225x; 1.3225x over previous
import functools

import jax
import jax.numpy as jnp
from jax import lax
from jax.experimental import pallas as pl
from jax.experimental.pallas import tpu as pltpu

TOPK = 8


def _router_body(x_ref, gw_ref, logits_ref, combine_ref):
    x = x_ref[...]
    gw = gw_ref[...]
    logits = lax.dot_general(x, gw, (((1,), (1,)), ((), ())),
                             preferred_element_type=jnp.float32)
    logits_ref[...] = logits
    m = jnp.max(logits, axis=1, keepdims=True)
    p = jnp.exp(logits - m)
    p = p / jnp.sum(p, axis=1, keepdims=True)
    E = p.shape[1]
    lane = lax.broadcasted_iota(jnp.int32, p.shape, 1)
    work = p
    combine = jnp.zeros_like(p)
    for _ in range(TOPK):
        mx = jnp.max(work, axis=1, keepdims=True)
        eq = work == mx
        first = jnp.min(jnp.where(eq, lane, E), axis=1, keepdims=True)
        oh = lane == first
        combine = jnp.where(oh, mx, combine)
        work = jnp.where(oh, -1.0, work)
    wsum = jnp.sum(combine, axis=1, keepdims=True)
    combine_ref[...] = combine / wsum


def _moe_dense_body(x_ref, wg_ref, wu_ref, wd_ref, comb_ref, out_ref):
    e = pl.program_id(0)
    x = x_ref[...]
    h = jnp.dot(x, wg_ref[0], preferred_element_type=jnp.float32)
    u = jnp.dot(x, wu_ref[0], preferred_element_type=jnp.float32)
    act = h * jax.nn.sigmoid(h) * u
    y = jnp.dot(act, wd_ref[0], preferred_element_type=jnp.float32)
    col = lax.broadcasted_iota(jnp.int32, (comb_ref.shape[1], 1), 0)
    oh = (col == e).astype(jnp.float32)
    w_col = jnp.dot(comb_ref[...], oh, preferred_element_type=jnp.float32)

    @pl.when(e == 0)
    def _():
        out_ref[...] = y * w_col

    @pl.when(e != 0)
    def _():
        out_ref[...] += y * w_col


def _shared_body(x_ref, shg_ref, shu_ref, shd_ref, segw_ref, moe_ref, out_ref,
                 *, nchunk):
    j = pl.program_id(0)
    x = x_ref[...]
    g = jnp.dot(x, shg_ref[...], preferred_element_type=jnp.float32)
    u = jnp.dot(x, shu_ref[...], preferred_element_type=jnp.float32)
    s = jnp.dot(g * jax.nn.sigmoid(g) * u, shd_ref[...],
                preferred_element_type=jnp.float32)

    @pl.when(j == 0)
    def _():
        out_ref[...] = s

    @pl.when(j != 0)
    def _():
        out_ref[...] += s

    @pl.when(j == nchunk - 1)
    def _():
        gate = jax.nn.sigmoid(jnp.dot(x, segw_ref[...],
                                      preferred_element_type=jnp.float32))
        out_ref[...] = moe_ref[...] + gate * out_ref[...]


def kernel(hidden_states, gate_w, W_gate, W_up, W_down, sh_gate_w, sh_up_w,
           sh_down_w, shared_expert_gate_w):
    b, s, d = hidden_states.shape
    t = b * s
    e, _, f = W_gate.shape
    sf = sh_gate_w.shape[1]
    x = hidden_states.reshape(t, d)

    logits, combine = pl.pallas_call(
        _router_body,
        out_shape=(
            jax.ShapeDtypeStruct((t, e), jnp.float32),
            jax.ShapeDtypeStruct((t, e), jnp.float32),
        ),
    )(x, gate_w)

    moe = pl.pallas_call(
        _moe_dense_body,
        grid=(e,),
        in_specs=[
            pl.BlockSpec((t, d), lambda i: (0, 0)),
            pl.BlockSpec((1, d, f), lambda i: (i, 0, 0)),
            pl.BlockSpec((1, d, f), lambda i: (i, 0, 0)),
            pl.BlockSpec((1, f, d), lambda i: (i, 0, 0)),
            pl.BlockSpec((t, e), lambda i: (0, 0)),
        ],
        out_specs=pl.BlockSpec((t, d), lambda i: (0, 0)),
        out_shape=jax.ShapeDtypeStruct((t, d), jnp.float32),
    )(x, W_gate, W_up, W_down, combine)

    nchunk = 11 if sf % 11 == 0 else 1
    cf = sf // nchunk
    out = pl.pallas_call(
        functools.partial(_shared_body, nchunk=nchunk),
        grid=(nchunk,),
        in_specs=[
            pl.BlockSpec((t, d), lambda j: (0, 0)),
            pl.BlockSpec((d, cf), lambda j: (0, j)),
            pl.BlockSpec((d, cf), lambda j: (0, j)),
            pl.BlockSpec((cf, d), lambda j: (j, 0)),
            pl.BlockSpec((d, 1), lambda j: (0, 0)),
            pl.BlockSpec((t, d), lambda j: (0, 0)),
        ],
        out_specs=pl.BlockSpec((t, d), lambda j: (0, 0)),
        out_shape=jax.ShapeDtypeStruct((t, d), jnp.float32),
    )(x, sh_gate_w, sh_up_w, sh_down_w, shared_expert_gate_w, moe)

    return (out.reshape(b, s, d), logits)
